# baseline (speedup 1.0000x reference)
import functools

import jax
import jax.numpy as jnp
from jax import lax
from jax.experimental import pallas as pl
from jax.experimental.pallas import tpu as pltpu
from jax.experimental.pallas import tpu_sc as plsc

_B = 128
_S = 8192
_N = _B * _S
_NW = 32

_K_TC = 112
_N_TC = _K_TC * _S
_N_SC = _N - _N_TC
_PER_W = _N_SC // _NW
_NCHUNK = 2
_C = _PER_W // _NCHUNK

_LANES = 128
_ROWS = _N // _LANES
_ROWS_TC = _N_TC // _LANES
_TC_GRID = 8
_TC_BLOCK = _ROWS // _TC_GRID

_C0 = 2.88538788
_C1 = 0.9620558
_C2 = 0.56891856
_C3 = 0.5052695


def _neg_log2(x, inv, den_other):
    bits = lax.bitcast_convert_type(x, jnp.int32)
    ke = 127 - lax.shift_right_logical(bits, 23)
    m_bits = lax.bitwise_or(lax.bitwise_and(bits, 0x007FFFFF), 0x3F800000)
    m = lax.bitcast_convert_type(m_bits, jnp.float32)
    s = (m - 1.0) * (den_other * inv)
    z = s * s
    p = _C0 + z * (_C1 + z * (_C2 + z * _C3))
    return ke.astype(jnp.float32) - s * p


def _mant_p1(x):
    bits = lax.bitcast_convert_type(x, jnp.int32)
    m_bits = lax.bitwise_or(lax.bitwise_and(bits, 0x007FFFFF), 0x3F800000)
    return lax.bitcast_convert_type(m_bits, jnp.float32) + 1.0


def _sample(l, a, b):
    den_a = _mant_p1(a)
    den_b = _mant_p1(b)
    inv = 1.0 / (den_a * den_b)
    ka = _neg_log2(a, inv, den_b)
    kb = _neg_log2(b, inv, den_a)
    t = jnp.exp(2.0 - 4.0 * l)
    bb = kb * kb
    return bb / (ka * ka * t + bb)


def _sc_body(l_hbm, ua_hbm, ub_hbm, out_hbm,
             lv, av, bv, ov, isem0, isem1, osem0, osem1):
    wid = lax.axis_index("s") * 2 + lax.axis_index("c")
    base = _N_TC + wid * _PER_W
    isems = (isem0, isem1)
    osems = (osem0, osem1)

    def start_in(c):
        p = c % 2
        off = base + c * _C
        return [
            pltpu.async_copy(l_hbm.at[pl.ds(off, _C)], lv.at[p], isems[p]),
            pltpu.async_copy(ua_hbm.at[pl.ds(off, _C)], av.at[p], isems[p]),
            pltpu.async_copy(ub_hbm.at[pl.ds(off, _C)], bv.at[p], isems[p]),
        ]

    in_h = {0: start_in(0)}
    out_h = {}
    for c in range(_NCHUNK):
        p = c % 2
        if c + 1 < _NCHUNK:
            in_h[c + 1] = start_in(c + 1)
        for h in in_h.pop(c):
            h.wait()
        if c - 2 in out_h:
            out_h.pop(c - 2).wait()

        @plsc.parallel_loop(0, _C, step=16, unroll=4)
        def body(i):
            ix = pl.ds(i, 16)
            ov[p, ix] = _sample(lv[p, ix], av[p, ix], bv[p, ix])

        out_h[c] = pltpu.async_copy(
            ov.at[p], out_hbm.at[pl.ds(base - _N_TC + c * _C, _C)], osems[p]
        )
    for c in sorted(out_h):
        out_h.pop(c).wait()


@functools.cache
def _sc_call():
    return pl.kernel(
        _sc_body,
        out_type=jax.ShapeDtypeStruct((_N_SC,), jnp.float32),
        mesh=plsc.VectorSubcoreMesh(core_axis_name="c", subcore_axis_name="s"),
        scratch_types=[
            pltpu.VMEM((2, _C), jnp.float32),
            pltpu.VMEM((2, _C), jnp.float32),
            pltpu.VMEM((2, _C), jnp.float32),
            pltpu.VMEM((2, _C), jnp.float32),
            pltpu.SemaphoreType.DMA,
            pltpu.SemaphoreType.DMA,
            pltpu.SemaphoreType.DMA,
            pltpu.SemaphoreType.DMA,
        ],
    )


def _tc_body(l_ref, a_ref, b_ref, o_ref):
    @pl.when(pl.program_id(0) * _TC_BLOCK < _ROWS_TC)
    def _():
        l = l_ref[...]
        la = -jnp.log(a_ref[...])
        lb = -jnp.log(b_ref[...])
        t = jnp.exp(2.0 - 4.0 * l)
        bb = lb * lb
        o_ref[...] = bb / (la * la * t + bb)


_N_BLK_TC = _ROWS_TC // _TC_BLOCK


@functools.cache
def _tc_call():
    in_spec = pl.BlockSpec(
        (_TC_BLOCK, _LANES), lambda i: (jnp.minimum(i, _N_BLK_TC - 1), 0)
    )
    out_spec = pl.BlockSpec((_TC_BLOCK, _LANES), lambda i: (i, 0))
    return pl.pallas_call(
        _tc_body,
        grid=(_TC_GRID,),
        in_specs=[in_spec, in_spec, in_spec],
        out_specs=out_spec,
        out_shape=jax.ShapeDtypeStruct((_ROWS, _LANES), jnp.float32),
    )


@jax.jit
def kernel(logits, uniform_a, uniform_b):
    l = logits.reshape(_N)
    ua = uniform_a.reshape(_N)
    ub = uniform_b.reshape(_N)
    l2 = l.reshape(_ROWS, _LANES)
    ua2 = ua.reshape(_ROWS, _LANES)
    ub2 = ub.reshape(_ROWS, _LANES)
    sc_out = _sc_call()(l, ua, ub)
    tc_out = _tc_call()(l2, ua2, ub2)
    out = jnp.concatenate([lax.slice(tc_out.reshape(_N), (0,), (_N_TC,)), sc_out])
    return out.reshape(_B, _S, 1)


# speedup vs baseline: 1.0699x; 1.0699x over previous
import functools

import jax
import jax.numpy as jnp
from jax import lax
from jax.experimental import pallas as pl
from jax.experimental.pallas import tpu as pltpu
from jax.experimental.pallas import tpu_sc as plsc

_B = 128
_S = 8192
_N = _B * _S
_NW = 32

_K_TC = 112
_N_TC = _K_TC * _S
_N_SC = _N - _N_TC
_PER_W = _N_SC // _NW
_NCHUNK = 2
_C = _PER_W // _NCHUNK

_LANES = 128
_ROWS = _N // _LANES
_ROWS_TC = _N_TC // _LANES
_TC_GRID = 8
_TC_BLOCK = _ROWS // _TC_GRID

_C0 = 2.88538788
_C1 = 0.9620558
_C2 = 0.56891856
_C3 = 0.5052695


def _neg_log2(x, inv, den_other):
    bits = lax.bitcast_convert_type(x, jnp.int32)
    ke = 127 - lax.shift_right_logical(bits, 23)
    m_bits = lax.bitwise_or(lax.bitwise_and(bits, 0x007FFFFF), 0x3F800000)
    m = lax.bitcast_convert_type(m_bits, jnp.float32)
    s = (m - 1.0) * (den_other * inv)
    z = s * s
    p = _C0 + z * (_C1 + z * (_C2 + z * _C3))
    return ke.astype(jnp.float32) - s * p


def _mant_p1(x):
    bits = lax.bitcast_convert_type(x, jnp.int32)
    m_bits = lax.bitwise_or(lax.bitwise_and(bits, 0x007FFFFF), 0x3F800000)
    return lax.bitcast_convert_type(m_bits, jnp.float32) + 1.0


def _sample(l, a, b):
    den_a = _mant_p1(a)
    den_b = _mant_p1(b)
    inv = 1.0 / (den_a * den_b)
    ka = _neg_log2(a, inv, den_b)
    kb = _neg_log2(b, inv, den_a)
    t = jnp.exp(2.0 - 4.0 * l)
    bb = kb * kb
    return bb / (ka * ka * t + bb)


def _sc_body(l_hbm, ua_hbm, ub_hbm, out_hbm,
             lv, av, bv, ov, isem0, isem1, osem0, osem1):
    wid = lax.axis_index("s") * 2 + lax.axis_index("c")
    base = _N_TC + wid * _PER_W
    isems = (isem0, isem1)
    osems = (osem0, osem1)

    def start_in(c):
        p = c % 2
        off = base + c * _C
        return [
            pltpu.async_copy(l_hbm.at[pl.ds(off, _C)], lv.at[p], isems[p]),
            pltpu.async_copy(ua_hbm.at[pl.ds(off, _C)], av.at[p], isems[p]),
            pltpu.async_copy(ub_hbm.at[pl.ds(off, _C)], bv.at[p], isems[p]),
        ]

    in_h = {0: start_in(0)}
    out_h = {}
    for c in range(_NCHUNK):
        p = c % 2
        if c + 1 < _NCHUNK:
            in_h[c + 1] = start_in(c + 1)
        for h in in_h.pop(c):
            h.wait()
        if c - 2 in out_h:
            out_h.pop(c - 2).wait()

        @plsc.parallel_loop(0, _C, step=16, unroll=8)
        def body(i):
            ix = pl.ds(i, 16)
            ov[p, ix] = _sample(lv[p, ix], av[p, ix], bv[p, ix])

        out_h[c] = pltpu.async_copy(
            ov.at[p], out_hbm.at[pl.ds(base - _N_TC + c * _C, _C)], osems[p]
        )
    for c in sorted(out_h):
        out_h.pop(c).wait()


@functools.cache
def _sc_call():
    return pl.kernel(
        _sc_body,
        out_type=jax.ShapeDtypeStruct((_N_SC,), jnp.float32),
        mesh=plsc.VectorSubcoreMesh(core_axis_name="c", subcore_axis_name="s"),
        scratch_types=[
            pltpu.VMEM((2, _C), jnp.float32),
            pltpu.VMEM((2, _C), jnp.float32),
            pltpu.VMEM((2, _C), jnp.float32),
            pltpu.VMEM((2, _C), jnp.float32),
            pltpu.SemaphoreType.DMA,
            pltpu.SemaphoreType.DMA,
            pltpu.SemaphoreType.DMA,
            pltpu.SemaphoreType.DMA,
        ],
    )


def _tc_body(l_ref, a_ref, b_ref, o_ref):
    @pl.when(pl.program_id(0) * _TC_BLOCK < _ROWS_TC)
    def _():
        l = l_ref[...]
        la = -jnp.log(a_ref[...])
        lb = -jnp.log(b_ref[...])
        t = jnp.exp(2.0 - 4.0 * l)
        bb = lb * lb
        o_ref[...] = bb / (la * la * t + bb)


_N_BLK_TC = _ROWS_TC // _TC_BLOCK


@functools.cache
def _tc_call():
    in_spec = pl.BlockSpec(
        (_TC_BLOCK, _LANES), lambda i: (jnp.minimum(i, _N_BLK_TC - 1), 0)
    )
    out_spec = pl.BlockSpec((_TC_BLOCK, _LANES), lambda i: (i, 0))
    return pl.pallas_call(
        _tc_body,
        grid=(_TC_GRID,),
        in_specs=[in_spec, in_spec, in_spec],
        out_specs=out_spec,
        out_shape=jax.ShapeDtypeStruct((_ROWS, _LANES), jnp.float32),
    )


@jax.jit
def kernel(logits, uniform_a, uniform_b):
    l = logits.reshape(_N)
    ua = uniform_a.reshape(_N)
    ub = uniform_b.reshape(_N)
    l2 = l.reshape(_ROWS, _LANES)
    ua2 = ua.reshape(_ROWS, _LANES)
    ub2 = ub.reshape(_ROWS, _LANES)
    sc_out = _sc_call()(l, ua, ub)
    tc_out = _tc_call()(l2, ua2, ub2)
    out = lax.dynamic_update_slice(tc_out.reshape(_N), sc_out, (_N_TC,))
    return out.reshape(_B, _S, 1)


# speedup vs baseline: 1.1030x; 1.0310x over previous
import functools

import jax
import jax.numpy as jnp
from jax import lax
from jax.experimental import pallas as pl
from jax.experimental.pallas import tpu as pltpu
from jax.experimental.pallas import tpu_sc as plsc

_B = 128
_S = 8192
_N = _B * _S
_NW = 32

_K_TC = 112
_N_TC = _K_TC * _S
_N_SC = _N - _N_TC
_PER_W = _N_SC // _NW
_NCHUNK = 2
_C = _PER_W // _NCHUNK

_LANES = 128
_ROWS = _N // _LANES
_ROWS_TC = _N_TC // _LANES
_TC_GRID = 8
_TC_BLOCK = _ROWS // _TC_GRID

_C0 = 2.88538788
_C1 = 0.9620558
_C2 = 0.56891856
_C3 = 0.5052695


def _neg_log2(x, inv, den_other):
    bits = lax.bitcast_convert_type(x, jnp.int32)
    ke = 127 - lax.shift_right_logical(bits, 23)
    m_bits = lax.bitwise_or(lax.bitwise_and(bits, 0x007FFFFF), 0x3F800000)
    m = lax.bitcast_convert_type(m_bits, jnp.float32)
    s = (m - 1.0) * (den_other * inv)
    z = s * s
    p = _C0 + z * (_C1 + z * (_C2 + z * _C3))
    return ke.astype(jnp.float32) - s * p


def _mant_p1(x):
    bits = lax.bitcast_convert_type(x, jnp.int32)
    m_bits = lax.bitwise_or(lax.bitwise_and(bits, 0x007FFFFF), 0x3F800000)
    return lax.bitcast_convert_type(m_bits, jnp.float32) + 1.0


def _sample(l, a, b):
    den_a = _mant_p1(a)
    den_b = _mant_p1(b)
    inv = 1.0 / (den_a * den_b)
    ka = _neg_log2(a, inv, den_b)
    kb = _neg_log2(b, inv, den_a)
    t = jnp.exp(2.0 - 4.0 * l)
    bb = kb * kb
    return bb / (ka * ka * t + bb)


def _sc_body(l_hbm, ua_hbm, ub_hbm, out_hbm,
             lv, av, bv, ov, isem0, isem1, osem0, osem1):
    wid = lax.axis_index("s") * 2 + lax.axis_index("c")
    base = _N_TC + wid * _PER_W
    isems = (isem0, isem1)
    osems = (osem0, osem1)

    def start_in(c):
        p = c % 2
        off = base + c * _C
        return [
            pltpu.async_copy(l_hbm.at[pl.ds(off, _C)], lv.at[p], isems[p]),
            pltpu.async_copy(ua_hbm.at[pl.ds(off, _C)], av.at[p], isems[p]),
            pltpu.async_copy(ub_hbm.at[pl.ds(off, _C)], bv.at[p], isems[p]),
        ]

    in_h = {0: start_in(0)}
    out_h = {}
    for c in range(_NCHUNK):
        p = c % 2
        if c + 1 < _NCHUNK:
            in_h[c + 1] = start_in(c + 1)
        for h in in_h.pop(c):
            h.wait()
        if c - 2 in out_h:
            out_h.pop(c - 2).wait()

        @plsc.parallel_loop(0, _C, step=16, unroll=4)
        def body(i):
            ix = pl.ds(i, 16)
            ov[p, ix] = _sample(lv[p, ix], av[p, ix], bv[p, ix])

        out_h[c] = pltpu.async_copy(
            ov.at[p], out_hbm.at[pl.ds(base - _N_TC + c * _C, _C)], osems[p]
        )
    for c in sorted(out_h):
        out_h.pop(c).wait()


@functools.cache
def _sc_call():
    return pl.kernel(
        _sc_body,
        out_type=jax.ShapeDtypeStruct((_N_SC,), jnp.float32),
        mesh=plsc.VectorSubcoreMesh(core_axis_name="c", subcore_axis_name="s"),
        scratch_types=[
            pltpu.VMEM((2, _C), jnp.float32),
            pltpu.VMEM((2, _C), jnp.float32),
            pltpu.VMEM((2, _C), jnp.float32),
            pltpu.VMEM((2, _C), jnp.float32),
            pltpu.SemaphoreType.DMA,
            pltpu.SemaphoreType.DMA,
            pltpu.SemaphoreType.DMA,
            pltpu.SemaphoreType.DMA,
        ],
    )


def _tc_body(l_ref, a_ref, b_ref, o_ref):
    @pl.when(pl.program_id(0) * _TC_BLOCK < _ROWS_TC)
    def _():
        l = l_ref[...]
        la = -jnp.log(a_ref[...])
        lb = -jnp.log(b_ref[...])
        t = jnp.exp(2.0 - 4.0 * l)
        bb = lb * lb
        o_ref[...] = bb / (la * la * t + bb)


_N_BLK_TC = _ROWS_TC // _TC_BLOCK


@functools.cache
def _tc_call():
    in_spec = pl.BlockSpec(
        (_TC_BLOCK, _LANES), lambda i: (jnp.minimum(i, _N_BLK_TC - 1), 0)
    )
    out_spec = pl.BlockSpec((_TC_BLOCK, _LANES), lambda i: (i, 0))
    return pl.pallas_call(
        _tc_body,
        grid=(_TC_GRID,),
        in_specs=[in_spec, in_spec, in_spec],
        out_specs=out_spec,
        out_shape=jax.ShapeDtypeStruct((_ROWS, _LANES), jnp.float32),
    )


@jax.jit
def kernel(logits, uniform_a, uniform_b):
    l = logits.reshape(_N)
    ua = uniform_a.reshape(_N)
    ub = uniform_b.reshape(_N)
    l2 = l.reshape(_ROWS, _LANES)
    ua2 = ua.reshape(_ROWS, _LANES)
    ub2 = ub.reshape(_ROWS, _LANES)
    sc_out = _sc_call()(l, ua, ub)
    tc_out = _tc_call()(l2, ua2, ub2)
    out = lax.dynamic_update_slice(tc_out.reshape(_N), sc_out, (_N_TC,))
    return out.reshape(_B, _S, 1)
